# Initial kernel scaffold; baseline (speedup 1.0000x reference)
import jax
import jax.numpy as jnp
from jax.experimental import pallas as pl


def kernel(x, table):
    raise NotImplementedError("write your pallas kernel here")


# speedup vs baseline: 5.8808x; 5.8808x over previous
import functools
import math

import jax
import jax.numpy as jnp
from jax import lax
from jax.experimental import pallas as pl
from jax.experimental.pallas import tpu as pltpu
from jax.experimental.pallas import tpu_sc as plsc

D_MODEL = 128
SCALE = math.sqrt(D_MODEL)

NC = 2
NS = 16
NW = NC * NS

ROWS_PER_CHUNK = 128


def _scale_body(t_ref, o_ref):
    o_ref[...] = t_ref[...] * SCALE


def _scale_table(table):
    v, d = table.shape
    blk = 4000
    grid = v // blk
    return pl.pallas_call(
        _scale_body,
        out_shape=jax.ShapeDtypeStruct((v, d), jnp.float32),
        grid=(grid,),
        in_specs=[pl.BlockSpec((blk, d), lambda i: (i, 0))],
        out_specs=pl.BlockSpec((blk, d), lambda i: (i, 0)),
    )(table)


def _make_gather(n_rows):
    chunks_total = n_rows // ROWS_PER_CHUNK
    cpw = chunks_total // NW
    mesh = plsc.VectorSubcoreMesh(core_axis_name="c", subcore_axis_name="s")

    @functools.partial(
        pl.kernel,
        out_type=jax.ShapeDtypeStruct((n_rows, D_MODEL), jnp.float32),
        mesh=mesh,
        scratch_types=[
            pltpu.VMEM((cpw, ROWS_PER_CHUNK), jnp.int32),
            pltpu.VMEM((ROWS_PER_CHUNK, D_MODEL), jnp.float32),
            pltpu.SemaphoreType.DMA,
        ],
    )
    def gather(table_hbm, idx_hbm, out_hbm, idx_v, rows_v, sem):
        wid = lax.axis_index("s") * NC + lax.axis_index("c")
        pltpu.sync_copy(idx_hbm.at[pl.ds(wid * cpw, cpw)], idx_v)

        def body(j, carry):
            pltpu.async_copy(table_hbm.at[idx_v.at[j]], rows_v, sem).wait()
            row0 = (wid * cpw + j) * ROWS_PER_CHUNK
            pltpu.sync_copy(rows_v, out_hbm.at[pl.ds(row0, ROWS_PER_CHUNK)])
            return carry

        lax.fori_loop(0, cpw, body, 0)

    return gather


@jax.jit
def kernel(x, table):
    scaled = _scale_table(table)
    n_rows = x.size
    xf = x.reshape(n_rows // ROWS_PER_CHUNK, ROWS_PER_CHUNK).astype(jnp.int32)
    out = _make_gather(n_rows)(scaled, xf)
    return out.reshape(x.shape + (D_MODEL,))


# speedup vs baseline: 8.2886x; 1.4094x over previous
import functools
import math

import jax
import jax.numpy as jnp
from jax import lax
from jax.experimental import pallas as pl
from jax.experimental.pallas import tpu as pltpu
from jax.experimental.pallas import tpu_sc as plsc

D_MODEL = 128
SCALE = math.sqrt(D_MODEL)

NC = 2
NS = 16
NW = NC * NS

ROWS_PER_CHUNK = 128


def _scale_body(t_ref, o_ref):
    o_ref[...] = t_ref[...] * SCALE


def _scale_table(table):
    v, d = table.shape
    blk = 4000
    grid = v // blk
    return pl.pallas_call(
        _scale_body,
        out_shape=jax.ShapeDtypeStruct((v, d), jnp.float32),
        grid=(grid,),
        in_specs=[pl.BlockSpec((blk, d), lambda i: (i, 0))],
        out_specs=pl.BlockSpec((blk, d), lambda i: (i, 0)),
    )(table)


NBUF = 4


def _make_gather(n_rows):
    chunks_total = n_rows // ROWS_PER_CHUNK
    cpw = chunks_total // NW
    assert cpw % NBUF == 0
    mesh = plsc.VectorSubcoreMesh(core_axis_name="c", subcore_axis_name="s")

    @functools.partial(
        pl.kernel,
        out_type=jax.ShapeDtypeStruct((n_rows, D_MODEL), jnp.float32),
        mesh=mesh,
        scratch_types=[
            pltpu.VMEM((cpw, ROWS_PER_CHUNK), jnp.int32),
            pltpu.VMEM((NBUF, ROWS_PER_CHUNK, D_MODEL), jnp.float32),
            [pltpu.SemaphoreType.DMA] * NBUF,
            [pltpu.SemaphoreType.DMA] * NBUF,
        ],
    )
    def gather(table_hbm, idx_hbm, out_hbm, idx_v, rows_v, gsems, psems):
        wid = lax.axis_index("s") * NC + lax.axis_index("c")
        pltpu.sync_copy(idx_hbm.at[pl.ds(wid * cpw, cpw)], idx_v)
        base = wid * cpw

        def start_gather(j, b):
            pltpu.async_copy(table_hbm.at[idx_v.at[j]], rows_v.at[b], gsems[b])

        def wait_gather(b):
            pltpu.make_async_copy(
                table_hbm.at[pl.ds(0, ROWS_PER_CHUNK)], rows_v.at[b], gsems[b]
            ).wait()

        def start_put(j, b):
            row0 = (base + j) * ROWS_PER_CHUNK
            pltpu.async_copy(
                rows_v.at[b], out_hbm.at[pl.ds(row0, ROWS_PER_CHUNK)], psems[b]
            )

        def wait_put(b):
            pltpu.make_async_copy(
                rows_v.at[b], out_hbm.at[pl.ds(0, ROWS_PER_CHUNK)], psems[b]
            ).wait()

        for b in range(NBUF):
            start_gather(b, b)

        def super_body(jj, carry):
            for b in range(NBUF):
                j = jj * NBUF + b
                wait_gather(b)
                start_put(j, b)
                last = jj == cpw // NBUF - 1

                @pl.when(jnp.logical_not(last))
                def _():
                    wait_put(b)
                    start_gather(j + NBUF, b)

            return carry

        lax.fori_loop(0, cpw // NBUF, super_body, 0)
        for b in range(NBUF):
            wait_put(b)

    return gather


@jax.jit
def kernel(x, table):
    scaled = _scale_table(table)
    n_rows = x.size
    xf = x.reshape(n_rows // ROWS_PER_CHUNK, ROWS_PER_CHUNK).astype(jnp.int32)
    out = _make_gather(n_rows)(scaled, xf)
    return out.reshape(x.shape + (D_MODEL,))


# speedup vs baseline: 8.3860x; 1.0118x over previous
import functools
import math

import jax
import jax.numpy as jnp
from jax import lax
from jax.experimental import pallas as pl
from jax.experimental.pallas import tpu as pltpu
from jax.experimental.pallas import tpu_sc as plsc

D_MODEL = 128
SCALE = math.sqrt(D_MODEL)

NC = 2
NS = 16
NW = NC * NS

ROWS_PER_CHUNK = 128


def _scale_body(t_ref, o_ref):
    o_ref[...] = t_ref[...] * SCALE


def _scale_table(table):
    v, d = table.shape
    blk = 10000
    grid = v // blk
    return pl.pallas_call(
        _scale_body,
        out_shape=jax.ShapeDtypeStruct((v, d), jnp.float32),
        grid=(grid,),
        in_specs=[pl.BlockSpec((blk, d), lambda i: (i, 0))],
        out_specs=pl.BlockSpec((blk, d), lambda i: (i, 0)),
    )(table)


NBUF = 5


def _make_gather(n_rows):
    chunks_total = n_rows // ROWS_PER_CHUNK
    cpw = chunks_total // NW
    assert cpw % NBUF == 0
    mesh = plsc.VectorSubcoreMesh(core_axis_name="c", subcore_axis_name="s")

    @functools.partial(
        pl.kernel,
        out_type=jax.ShapeDtypeStruct((n_rows, D_MODEL), jnp.float32),
        mesh=mesh,
        scratch_types=[
            pltpu.VMEM((cpw, ROWS_PER_CHUNK), jnp.int32),
            pltpu.VMEM((NBUF, ROWS_PER_CHUNK, D_MODEL), jnp.float32),
            [pltpu.SemaphoreType.DMA] * NBUF,
            [pltpu.SemaphoreType.DMA] * NBUF,
        ],
    )
    def gather(table_hbm, idx_hbm, out_hbm, idx_v, rows_v, gsems, psems):
        wid = lax.axis_index("s") * NC + lax.axis_index("c")
        pltpu.sync_copy(idx_hbm.at[pl.ds(wid * cpw, cpw)], idx_v)
        base = wid * cpw

        def start_gather(j, b):
            pltpu.async_copy(table_hbm.at[idx_v.at[j]], rows_v.at[b], gsems[b])

        def wait_gather(b):
            pltpu.make_async_copy(
                table_hbm.at[pl.ds(0, ROWS_PER_CHUNK)], rows_v.at[b], gsems[b]
            ).wait()

        def start_put(j, b):
            row0 = (base + j) * ROWS_PER_CHUNK
            pltpu.async_copy(
                rows_v.at[b], out_hbm.at[pl.ds(row0, ROWS_PER_CHUNK)], psems[b]
            )

        def wait_put(b):
            pltpu.make_async_copy(
                rows_v.at[b], out_hbm.at[pl.ds(0, ROWS_PER_CHUNK)], psems[b]
            ).wait()

        for b in range(NBUF):
            start_gather(b, b)

        def super_body(jj, carry):
            for b in range(NBUF):
                j = jj * NBUF + b
                wait_gather(b)
                start_put(j, b)
                last = jj == cpw // NBUF - 1

                @pl.when(jnp.logical_not(last))
                def _():
                    wait_put(b)
                    start_gather(j + NBUF, b)

            return carry

        lax.fori_loop(0, cpw // NBUF, super_body, 0)
        for b in range(NBUF):
            wait_put(b)

    return gather


@jax.jit
def kernel(x, table):
    scaled = _scale_table(table)
    n_rows = x.size
    xf = x.reshape(n_rows // ROWS_PER_CHUNK, ROWS_PER_CHUNK).astype(jnp.int32)
    out = _make_gather(n_rows)(scaled, xf)
    return out.reshape(x.shape + (D_MODEL,))
